# baseline (speedup 1.0000x reference)
import functools

import numpy as np
import jax
import jax.numpy as jnp
from jax import lax
from jax.experimental import pallas as pl
from jax.experimental.pallas import tpu as pltpu
from jax.experimental.pallas import tpu_sc as plsc

MAX_LEN = 200
MODEL_DIM = 64
NW = 32
CH = 40
LANES = 16


def _pe_np():
    pos = np.arange(MAX_LEN)[:, None]
    pe = pos / np.power(10000, 2.0 * np.arange(MODEL_DIM)[None, :] / MODEL_DIM)
    pe[:, 0::2] = np.sin(pe[:, 0::2])
    pe[:, 1::2] = np.cos(pe[:, 1::2])
    return pe.astype(np.float32)


def _sc_body(cpw, x2_hbm, pe_hbm, table_hbm, out_hbm, idx_all, pe_v, rows_v, gsem):
    wid = lax.axis_index("s") * 2 + lax.axis_index("c")
    pltpu.sync_copy(x2_hbm.at[pl.ds(wid * cpw, cpw)], idx_all)
    pltpu.sync_copy(pe_hbm, pe_v)

    def chunk_body(c, carry):
        pltpu.async_copy(table_hbm.at[idx_all.at[c]], rows_v, gsem).wait()
        p0 = (c % (MAX_LEN // CH)) * CH

        def add_body(r, carry2):
            for cc in range(MODEL_DIM // LANES):
                sl = pl.ds(cc * LANES, LANES)
                rows_v[r, sl] = rows_v[r, sl] + pe_v[p0 + r, sl]
            return carry2

        lax.fori_loop(0, CH, add_body, 0)
        pltpu.sync_copy(rows_v, out_hbm.at[pl.ds((wid * cpw + c) * CH, CH)])
        return carry

    lax.fori_loop(0, cpw, chunk_body, 0)


def kernel(x, table):
    b, seq = x.shape
    n_rows = b * seq
    assert seq == MAX_LEN and n_rows % (CH * NW) == 0
    cpw = n_rows // CH // NW
    x2 = x.reshape(n_rows // CH, CH).astype(jnp.int32)
    pe = jnp.asarray(_pe_np())

    mesh = plsc.VectorSubcoreMesh(core_axis_name="c", subcore_axis_name="s")
    k = functools.partial(
        pl.kernel,
        mesh=mesh,
        out_type=jax.ShapeDtypeStruct((n_rows, MODEL_DIM), jnp.float32),
        scratch_types=[
            pltpu.VMEM((cpw, CH), jnp.int32),
            pltpu.VMEM((MAX_LEN, MODEL_DIM), jnp.float32),
            pltpu.VMEM((CH, MODEL_DIM), jnp.float32),
            pltpu.SemaphoreType.DMA,
        ],
        compiler_params=pltpu.CompilerParams(use_tc_tiling_on_sc=False),
    )(functools.partial(_sc_body, cpw))
    out = k(x2, pe, table)
    return out.reshape(b, seq, MODEL_DIM)


# speedup vs baseline: 1.7268x; 1.7268x over previous
import functools

import numpy as np
import jax
import jax.numpy as jnp
from jax import lax
from jax.experimental import pallas as pl
from jax.experimental.pallas import tpu as pltpu
from jax.experimental.pallas import tpu_sc as plsc

MAX_LEN = 200
MODEL_DIM = 64
NW = 32
NB = 4
LANES = 16


def _pe_np():
    pos = np.arange(MAX_LEN)[:, None]
    pe = pos / np.power(10000, 2.0 * np.arange(MODEL_DIM)[None, :] / MODEL_DIM)
    pe[:, 0::2] = np.sin(pe[:, 0::2])
    pe[:, 1::2] = np.cos(pe[:, 1::2])
    return pe.astype(np.float32)


def _sc_body(cpw, x_hbm, pe_hbm, table_hbm, out_hbm, idx_all, pe_v,
             r0, r1, r2, r3, gs0, gs1, gs2, gs3, os0, os1, os2, os3):
    rows = (r0, r1, r2, r3)
    gsem = (gs0, gs1, gs2, gs3)
    osem = (os0, os1, os2, os3)
    wid = lax.axis_index("s") * 2 + lax.axis_index("c")
    base = wid * cpw
    pltpu.sync_copy(x_hbm.at[pl.ds(base, cpw)], idx_all)
    pltpu.sync_copy(pe_hbm, pe_v)

    def fire_gather(c, sl):
        pltpu.async_copy(table_hbm.at[idx_all.at[c]], rows[sl], gsem[sl])

    def wait_gather(c, sl):
        pltpu.make_async_copy(table_hbm.at[idx_all.at[c]], rows[sl],
                              gsem[sl]).wait()

    def fire_out(c, sl):
        pltpu.async_copy(rows[sl], out_hbm.at[pl.ds((base + c) * MAX_LEN,
                                                    MAX_LEN)], osem[sl])

    def wait_out(c, sl):
        pltpu.make_async_copy(rows[sl],
                              out_hbm.at[pl.ds((base + c) * MAX_LEN,
                                               MAX_LEN)], osem[sl]).wait()

    fire_gather(0, 0)
    fire_gather(1, 1)

    def outer(k, carry):
        for sl in range(NB):
            c = k * NB + sl
            wait_gather(c, sl)
            rv = rows[sl]

            def add_rows(r2i, carry2):
                r = r2i * 2
                for dr in range(2):
                    for cc in range(MODEL_DIM // LANES):
                        s = pl.ds(cc * LANES, LANES)
                        rv[r + dr, s] = rv[r + dr, s] + pe_v[r + dr, s]
                return carry2

            lax.fori_loop(0, MAX_LEN // 2, add_rows, 0)
            fire_out(c, sl)

            nsl = (sl + 2) % NB

            @pl.when(c + 2 < cpw)
            def _():
                @pl.when(c >= 2)
                def _():
                    wait_out(c - 2, nsl)
                fire_gather(c + 2, nsl)
        return carry

    lax.fori_loop(0, cpw // NB, outer, 0)
    for sl in range(NB):
        wait_out(cpw - NB + sl, (cpw - NB + sl) % NB)


def kernel(x, table):
    b, seq = x.shape
    assert seq == MAX_LEN and b % NW == 0
    cpw = b // NW
    xi = x.astype(jnp.int32)
    pe = jnp.asarray(_pe_np())

    mesh = plsc.VectorSubcoreMesh(core_axis_name="c", subcore_axis_name="s")
    k = functools.partial(
        pl.kernel,
        mesh=mesh,
        out_type=jax.ShapeDtypeStruct((b * seq, MODEL_DIM), jnp.float32),
        scratch_types=[
            pltpu.VMEM((cpw, MAX_LEN), jnp.int32),
            pltpu.VMEM((MAX_LEN, MODEL_DIM), jnp.float32),
        ] + [pltpu.VMEM((MAX_LEN, MODEL_DIM), jnp.float32)] * NB
          + [pltpu.SemaphoreType.DMA] * (2 * NB),
        compiler_params=pltpu.CompilerParams(use_tc_tiling_on_sc=False),
    )(functools.partial(_sc_body, cpw))
    out = k(xi, pe, table)
    return out.reshape(b, seq, MODEL_DIM)
